# baseline (speedup 1.0000x reference)
import functools

import jax
import jax.numpy as jnp
from jax import lax
from jax.experimental import pallas as pl
from jax.experimental.pallas import tpu as pltpu
from jax.experimental.pallas import tpu_sc as plsc

F32 = jnp.float32

_NC = 2
_NS = 16
_IB = 128


def _dot(a, b):
    return jnp.dot(a, b, preferred_element_type=F32)


def _embed_body(x_ref, w1_ref, b1_ref, w2_ref, b2_ref, o_ref):
    h = jnp.maximum(_dot(x_ref[...], w1_ref[...]) + b1_ref[...], 0.0)
    o_ref[...] = _dot(h, w2_ref[...]) + b2_ref[...]


def _embed(x, w1, b1, w2, b2, blk):
    n, din = x.shape
    dh = w1.shape[1]
    do = w2.shape[1]
    grid = n // blk
    return pl.pallas_call(
        _embed_body,
        grid=(grid,),
        in_specs=[
            pl.BlockSpec((blk, din), lambda i: (i, 0)),
            pl.BlockSpec((din, dh), lambda i: (0, 0)),
            pl.BlockSpec((1, dh), lambda i: (0, 0)),
            pl.BlockSpec((dh, do), lambda i: (0, 0)),
            pl.BlockSpec((1, do), lambda i: (0, 0)),
        ],
        out_specs=pl.BlockSpec((blk, do), lambda i: (i, 0)),
        out_shape=jax.ShapeDtypeStruct((n, do), F32),
    )(x, w1, b1.reshape(1, dh), w2, b2.reshape(1, do))


def _proj2_body(x_ref, ws_ref, wr_ref, ts_ref, tr_ref):
    x = x_ref[...]
    ts_ref[...] = _dot(x, ws_ref[...])
    tr_ref[...] = _dot(x, wr_ref[...])


def _proj2(x, ws, wr, blk, npad):
    n, d = x.shape
    do = ws.shape[1]
    return pl.pallas_call(
        _proj2_body,
        grid=(n // blk,),
        in_specs=[
            pl.BlockSpec((blk, d), lambda i: (i, 0)),
            pl.BlockSpec((d, do), lambda i: (0, 0)),
            pl.BlockSpec((d, do), lambda i: (0, 0)),
        ],
        out_specs=[
            pl.BlockSpec((blk, do), lambda i: (i, 0)),
            pl.BlockSpec((blk, do), lambda i: (i, 0)),
        ],
        out_shape=[
            jax.ShapeDtypeStruct((npad, do), F32),
            jax.ShapeDtypeStruct((npad, do), F32),
        ],
    )(x, ws, wr)


def _edge_mlp_body_res(e_ref, ga_ref, gb_ref, w1_ref, b1_ref, w2_ref, b2_ref,
                       ne_ref, eo_ref):
    h = jnp.maximum(_dot(e_ref[...], w1_ref[...]) + ga_ref[...] + gb_ref[...]
                    + b1_ref[...], 0.0)
    ne = _dot(h, w2_ref[...]) + b2_ref[...]
    ne_ref[...] = ne
    eo_ref[...] = e_ref[...] + ne


def _edge_mlp_body(e_ref, ga_ref, gb_ref, w1_ref, b1_ref, w2_ref, b2_ref,
                   ne_ref):
    h = jnp.maximum(_dot(e_ref[...], w1_ref[...]) + ga_ref[...] + gb_ref[...]
                    + b1_ref[...], 0.0)
    ne_ref[...] = _dot(h, w2_ref[...]) + b2_ref[...]


def _edge_mlp(e, ga, gb, w1, b1, w2, b2, blk, want_residual, epad):
    n, d = e.shape
    dh = w1.shape[1]
    do = w2.shape[1]
    in_specs = [
        pl.BlockSpec((blk, d), lambda i: (i, 0)),
        pl.BlockSpec((blk, dh), lambda i: (i, 0)),
        pl.BlockSpec((blk, dh), lambda i: (i, 0)),
        pl.BlockSpec((d, dh), lambda i: (0, 0)),
        pl.BlockSpec((1, dh), lambda i: (0, 0)),
        pl.BlockSpec((dh, do), lambda i: (0, 0)),
        pl.BlockSpec((1, do), lambda i: (0, 0)),
    ]
    args = (e, ga, gb, w1, b1.reshape(1, dh), w2, b2.reshape(1, do))
    if want_residual:
        ne, eo = pl.pallas_call(
            _edge_mlp_body_res,
            grid=(n // blk,),
            in_specs=in_specs,
            out_specs=[
                pl.BlockSpec((blk, do), lambda i: (i, 0)),
                pl.BlockSpec((blk, do), lambda i: (i, 0)),
            ],
            out_shape=[
                jax.ShapeDtypeStruct((epad, do), F32),
                jax.ShapeDtypeStruct((n, do), F32),
            ],
        )(*args)
        return ne, eo
    ne = pl.pallas_call(
        _edge_mlp_body,
        grid=(n // blk,),
        in_specs=in_specs,
        out_specs=pl.BlockSpec((blk, do), lambda i: (i, 0)),
        out_shape=jax.ShapeDtypeStruct((epad, do), F32),
    )(*args)
    return ne, None


def _node_mlp_body(x_ref, a_ref, w1a_ref, w1b_ref, b1_ref, w2_ref, b2_ref,
                   o_ref):
    h = jnp.maximum(
        _dot(x_ref[...], w1a_ref[...]) + _dot(a_ref[...], w1b_ref[...])
        + b1_ref[...], 0.0)
    o_ref[...] = x_ref[...] + _dot(h, w2_ref[...]) + b2_ref[...]


def _node_mlp(x, agg, w1a, w1b, b1, w2, b2, blk):
    n, d = x.shape
    dh = w1a.shape[1]
    do = w2.shape[1]
    return pl.pallas_call(
        _node_mlp_body,
        grid=(n // blk,),
        in_specs=[
            pl.BlockSpec((blk, d), lambda i: (i, 0)),
            pl.BlockSpec((blk, d), lambda i: (i, 0)),
            pl.BlockSpec((d, dh), lambda i: (0, 0)),
            pl.BlockSpec((d, dh), lambda i: (0, 0)),
            pl.BlockSpec((1, dh), lambda i: (0, 0)),
            pl.BlockSpec((dh, do), lambda i: (0, 0)),
            pl.BlockSpec((1, do), lambda i: (0, 0)),
        ],
        out_specs=pl.BlockSpec((blk, do), lambda i: (i, 0)),
        out_shape=jax.ShapeDtypeStruct((n, do), F32),
    )(x, agg, w1a, w1b, b1.reshape(1, dh), w2, b2.reshape(1, do))


def _combine_body(a_ref, ma_ref, b_ref, mb_ref, o_ref):
    o_ref[...] = a_ref[...] * ma_ref[...] + b_ref[...] * mb_ref[...]


def _combine(a, ma, b, mb, blk):
    n, d = a.shape
    return pl.pallas_call(
        _combine_body,
        grid=(n // blk,),
        in_specs=[
            pl.BlockSpec((blk, d), lambda i: (i, 0)),
            pl.BlockSpec((blk, 1), lambda i: (i, 0)),
            pl.BlockSpec((blk, d), lambda i: (i, 0)),
            pl.BlockSpec((blk, 1), lambda i: (i, 0)),
        ],
        out_specs=pl.BlockSpec((blk, d), lambda i: (i, 0)),
        out_shape=jax.ShapeDtypeStruct((n, d), F32),
    )(a, ma, b, mb)


def _final_body(ln_ref, pp_ref, wm_ref, pm_ref, p0_ref, w1_ref, b1_ref,
                w2_ref, b2_ref, o_ref):
    wm = wm_ref[...]
    pm = pm_ref[...]
    dec_in = ln_ref[...] * wm + pp_ref[...] * pm
    h = jnp.maximum(_dot(dec_in, w1_ref[...]) + b1_ref[...], 0.0)
    delta = _dot(h, w2_ref[...]) + b2_ref[...]
    o_ref[...] = (p0_ref[...] + delta) * (wm + pm)


def _final(ln, pp, wm, pm, p0, w1, b1, w2, b2, blk):
    n, d = ln.shape
    dh = w1.shape[1]
    do = w2.shape[1]
    return pl.pallas_call(
        _final_body,
        grid=(n // blk,),
        in_specs=[
            pl.BlockSpec((blk, d), lambda i: (i, 0)),
            pl.BlockSpec((blk, d), lambda i: (i, 0)),
            pl.BlockSpec((blk, 1), lambda i: (i, 0)),
            pl.BlockSpec((blk, 1), lambda i: (i, 0)),
            pl.BlockSpec((blk, 1), lambda i: (i, 0)),
            pl.BlockSpec((d, dh), lambda i: (0, 0)),
            pl.BlockSpec((1, dh), lambda i: (0, 0)),
            pl.BlockSpec((dh, do), lambda i: (0, 0)),
            pl.BlockSpec((1, do), lambda i: (0, 0)),
        ],
        out_specs=pl.BlockSpec((blk, do), lambda i: (i, 0)),
        out_shape=jax.ShapeDtypeStruct((n, do), F32),
    )(ln, pp, wm, pm, p0, w1, b1.reshape(1, dh), w2, b2.reshape(1, do))


def _gather_pair(table_s, table_r, s2d, r2d):
    nrows = s2d.shape[0]
    lat = table_s.shape[1]
    gch = 2
    inner = 4
    blkrows = gch * inner
    rows_pw = nrows // (_NC * _NS)
    nouter = rows_pw // blkrows
    cw = gch * _IB

    @functools.partial(
        pl.kernel,
        out_type=[jax.ShapeDtypeStruct((nrows * _IB, lat), F32),
                  jax.ShapeDtypeStruct((nrows * _IB, lat), F32)],
        mesh=plsc.VectorSubcoreMesh(core_axis_name="c", subcore_axis_name="s"),
        scratch_types=[pltpu.VMEM((blkrows, _IB), jnp.int32),
                       pltpu.VMEM((blkrows, _IB), jnp.int32),
                       pltpu.VMEM((cw, lat), F32),
                       pltpu.VMEM((cw, lat), F32),
                       pltpu.VMEM((cw, lat), F32),
                       pltpu.VMEM((cw, lat), F32),
                       pltpu.SemaphoreType.DMA,
                       pltpu.SemaphoreType.DMA],
        compiler_params=pltpu.CompilerParams(use_tc_tiling_on_sc=False),
    )
    def gk(ts_hbm, tr_hbm, s_hbm, r_hbm, outs_hbm, outr_hbm,
           sidx, ridx, bs0, br0, bs1, br1, sem0, sem1):
        wid = lax.axis_index("c") * _NS + lax.axis_index("s")
        row0 = wid * rows_pw
        bufs = [(bs0, br0, sem0), (bs1, br1, sem1)]

        def outer(o, carry):
            ro = row0 + o * blkrows
            pltpu.sync_copy(s_hbm.at[pl.ds(ro, blkrows)], sidx)
            pltpu.sync_copy(r_hbm.at[pl.ds(ro, blkrows)], ridx)

            def issue(c):
                bs, br, sem = bufs[c % 2]
                cps = []
                for k in range(gch):
                    cps.append(pltpu.async_copy(
                        ts_hbm.at[sidx.at[c * gch + k]],
                        bs.at[pl.ds(k * _IB, _IB)], sem))
                    cps.append(pltpu.async_copy(
                        tr_hbm.at[ridx.at[c * gch + k]],
                        br.at[pl.ds(k * _IB, _IB)], sem))
                return cps

            def drain(c, cps):
                bs, br, _ = bufs[c % 2]
                for cp in cps:
                    cp.wait()
                e0 = (ro + c * gch) * _IB
                pltpu.sync_copy(bs, outs_hbm.at[pl.ds(e0, cw)])
                pltpu.sync_copy(br, outr_hbm.at[pl.ds(e0, cw)])

            pend = [issue(0), issue(1)]
            for c in range(2, inner):
                drain(c - 2, pend[0])
                pend = [pend[1], issue(c)]
            drain(inner - 2, pend[0])
            drain(inner - 1, pend[1])
            return carry

        lax.fori_loop(0, nouter, outer, 0)

    return gk(table_s, table_r, s2d, r2d)


def _segment_sum_sc(ne, r2d, zeros, n_nodes, n_acc):
    nrows = r2d.shape[0]
    lat = ne.shape[1]
    half = lat // _NC
    sch = 4
    rows_ps = nrows // _NS
    nchunk = rows_ps // sch
    zrows = n_acc // _NS
    orows = n_nodes // _NS

    @functools.partial(
        pl.kernel,
        out_type=jax.ShapeDtypeStruct((n_nodes, lat), F32),
        mesh=plsc.VectorSubcoreMesh(core_axis_name="c", subcore_axis_name="s"),
        scratch_types=[pltpu.VMEM((sch, _IB), jnp.int32),
                       pltpu.VMEM((sch * _IB, half), F32),
                       pltpu.VMEM_SHARED((n_acc, half), F32)],
        compiler_params=pltpu.CompilerParams(use_tc_tiling_on_sc=False),
    )
    def sk(ne_hbm, r_hbm, z_hbm, out_hbm, ridx, vals, acc):
        c = lax.axis_index("c")
        s = lax.axis_index("s")
        col0 = c * half
        pltpu.sync_copy(z_hbm.at[pl.ds(s * zrows, zrows)],
                        acc.at[pl.ds(s * zrows, zrows)])
        plsc.subcore_barrier()

        def chunk(j, carry):
            r0 = s * rows_ps + j * sch
            e0 = r0 * _IB
            pltpu.sync_copy(r_hbm.at[pl.ds(r0, sch)], ridx)
            pltpu.sync_copy(ne_hbm.at[pl.ds(e0, sch * _IB), pl.ds(col0, half)],
                            vals)
            for k in range(sch):
                pltpu.sync_copy(vals.at[pl.ds(k * _IB, _IB)],
                                acc.at[ridx.at[k]], add=True)
            return carry

        lax.fori_loop(0, nchunk, chunk, 0)
        plsc.subcore_barrier()
        pltpu.sync_copy(acc.at[pl.ds(s * orows, orows)],
                        out_hbm.at[pl.ds(s * orows, orows),
                                   pl.ds(col0, half)])

    return sk(ne, r2d, zeros)


def _pad_idx(idx, pad_val, epad):
    pad = jnp.full((epad - idx.shape[0],), pad_val, jnp.int32)
    return jnp.concatenate([idx, pad]).reshape(epad // _IB, _IB)


def _gnn_steps(step_params, nodes, edges, s2d, r2d, zeros, nblk, eblk,
               n_acc, epad):
    n_nodes = nodes.shape[0]
    lat = nodes.shape[1]
    nsteps = len(step_params)
    for i, sp in enumerate(step_params):
        (w1e, b1e), (w2e, b2e) = sp['edge']
        (w1n, b1n), (w2n, b2n) = sp['node']
        gs, gr = _proj2(nodes, w1e[lat:2 * lat], w1e[2 * lat:], nblk, n_acc)
        gsg, grg = _gather_pair(gs, gr, s2d, r2d)
        ne, edges_next = _edge_mlp(edges, gsg, grg, w1e[:lat], b1e, w2e, b2e,
                                   eblk, want_residual=(i + 1 < nsteps),
                                   epad=epad)
        agg = _segment_sum_sc(ne, r2d, zeros, n_nodes, n_acc)
        nodes = _node_mlp(nodes, agg, w1n[:lat], w1n[lat:], b1n, w2n, b2n,
                          nblk)
        edges = edges_next
    return nodes


def kernel(nodes, edges, probe_nodes, probe_edges, wt_mask, probe_mask,
           params, senders, receivers, probe_senders, probe_receivers):
    n_nodes = nodes.shape[0]
    n_edges = edges.shape[0]
    lat = params['embed_node'][-1][0].shape[1]
    nblk = 2000 if n_nodes % 2000 == 0 else n_nodes
    eblk = 2000 if n_edges % 2000 == 0 else n_edges
    align = _IB * _NC * _NS * 8
    epad = -(-n_edges // align) * align
    n_acc = -(-(n_nodes + 1) // _NS) * _NS
    s2d = _pad_idx(senders, n_nodes, epad)
    r2d = _pad_idx(receivers, n_nodes, epad)
    ps2d = _pad_idx(probe_senders, n_nodes, epad)
    pr2d = _pad_idx(probe_receivers, n_nodes, epad)
    zeros = jnp.zeros((n_acc, lat // _NC), F32)

    (enw1, enb1), (enw2, enb2) = params['embed_node']
    (eew1, eeb1), (eew2, eeb2) = params['embed_edge']
    n_lat = _embed(nodes, enw1, enb1, enw2, enb2, nblk)
    e_lat = _embed(edges, eew1, eeb1, eew2, eeb2, eblk)
    pe_lat = _embed(probe_edges, eew1, eeb1, eew2, eeb2, eblk)

    wt_out = _gnn_steps(params['wt_gnn'], n_lat, e_lat, s2d, r2d, zeros,
                        nblk, eblk, n_acc, epad)
    latentspace = _combine(wt_out, wt_mask, n_lat, probe_mask, nblk)

    probe_out = _gnn_steps(params['probe_gnn'], latentspace, pe_lat,
                           ps2d, pr2d, zeros, nblk, eblk, n_acc, epad)

    (dw1, db1), (dw2, db2) = params['decoder']
    return _final(latentspace, probe_out, wt_mask, probe_mask,
                  probe_nodes[:, 0:1], dw1, db1, dw2, db2, nblk)


# speedup vs baseline: 1.0936x; 1.0936x over previous
import functools

import jax
import jax.numpy as jnp
from jax import lax
from jax.experimental import pallas as pl
from jax.experimental.pallas import tpu as pltpu
from jax.experimental.pallas import tpu_sc as plsc

F32 = jnp.float32

_NC = 2
_NS = 16
_IB = 128


def _dot(a, b):
    return jnp.dot(a, b, preferred_element_type=F32)


def _embed_body(x_ref, w1_ref, b1_ref, w2_ref, b2_ref, o_ref):
    h = jnp.maximum(_dot(x_ref[...], w1_ref[...]) + b1_ref[...], 0.0)
    o_ref[...] = _dot(h, w2_ref[...]) + b2_ref[...]


def _embed(x, w1, b1, w2, b2, blk):
    n, din = x.shape
    dh = w1.shape[1]
    do = w2.shape[1]
    grid = n // blk
    return pl.pallas_call(
        _embed_body,
        grid=(grid,),
        in_specs=[
            pl.BlockSpec((blk, din), lambda i: (i, 0)),
            pl.BlockSpec((din, dh), lambda i: (0, 0)),
            pl.BlockSpec((1, dh), lambda i: (0, 0)),
            pl.BlockSpec((dh, do), lambda i: (0, 0)),
            pl.BlockSpec((1, do), lambda i: (0, 0)),
        ],
        out_specs=pl.BlockSpec((blk, do), lambda i: (i, 0)),
        out_shape=jax.ShapeDtypeStruct((n, do), F32),
    )(x, w1, b1.reshape(1, dh), w2, b2.reshape(1, do))


def _proj2_body(x_ref, ws_ref, wr_ref, ts_ref, tr_ref):
    x = x_ref[...]
    ts_ref[...] = _dot(x, ws_ref[...])
    tr_ref[...] = _dot(x, wr_ref[...])


def _proj2(x, ws, wr, blk, npad):
    n, d = x.shape
    do = ws.shape[1]
    return pl.pallas_call(
        _proj2_body,
        grid=(n // blk,),
        in_specs=[
            pl.BlockSpec((blk, d), lambda i: (i, 0)),
            pl.BlockSpec((d, do), lambda i: (0, 0)),
            pl.BlockSpec((d, do), lambda i: (0, 0)),
        ],
        out_specs=[
            pl.BlockSpec((blk, do), lambda i: (i, 0)),
            pl.BlockSpec((blk, do), lambda i: (i, 0)),
        ],
        out_shape=[
            jax.ShapeDtypeStruct((npad, do), F32),
            jax.ShapeDtypeStruct((npad, do), F32),
        ],
    )(x, ws, wr)


def _edge_mlp_body_res(e_ref, ga_ref, gb_ref, w1_ref, b1_ref, w2_ref, b2_ref,
                       ne_ref, eo_ref):
    h = jnp.maximum(_dot(e_ref[...], w1_ref[...]) + ga_ref[...] + gb_ref[...]
                    + b1_ref[...], 0.0)
    ne = _dot(h, w2_ref[...]) + b2_ref[...]
    ne_ref[...] = ne
    eo_ref[...] = e_ref[...] + ne


def _edge_mlp_body(e_ref, ga_ref, gb_ref, w1_ref, b1_ref, w2_ref, b2_ref,
                   ne_ref):
    h = jnp.maximum(_dot(e_ref[...], w1_ref[...]) + ga_ref[...] + gb_ref[...]
                    + b1_ref[...], 0.0)
    ne_ref[...] = _dot(h, w2_ref[...]) + b2_ref[...]


def _edge_mlp(e, ga, gb, w1, b1, w2, b2, blk, want_residual, epad):
    n, d = e.shape
    dh = w1.shape[1]
    do = w2.shape[1]
    in_specs = [
        pl.BlockSpec((blk, d), lambda i: (i, 0)),
        pl.BlockSpec((blk, dh), lambda i: (i, 0)),
        pl.BlockSpec((blk, dh), lambda i: (i, 0)),
        pl.BlockSpec((d, dh), lambda i: (0, 0)),
        pl.BlockSpec((1, dh), lambda i: (0, 0)),
        pl.BlockSpec((dh, do), lambda i: (0, 0)),
        pl.BlockSpec((1, do), lambda i: (0, 0)),
    ]
    args = (e, ga, gb, w1, b1.reshape(1, dh), w2, b2.reshape(1, do))
    if want_residual:
        ne, eo = pl.pallas_call(
            _edge_mlp_body_res,
            grid=(n // blk,),
            in_specs=in_specs,
            out_specs=[
                pl.BlockSpec((blk, do), lambda i: (i, 0)),
                pl.BlockSpec((blk, do), lambda i: (i, 0)),
            ],
            out_shape=[
                jax.ShapeDtypeStruct((epad, do), F32),
                jax.ShapeDtypeStruct((n, do), F32),
            ],
        )(*args)
        return ne, eo
    ne = pl.pallas_call(
        _edge_mlp_body,
        grid=(n // blk,),
        in_specs=in_specs,
        out_specs=pl.BlockSpec((blk, do), lambda i: (i, 0)),
        out_shape=jax.ShapeDtypeStruct((epad, do), F32),
    )(*args)
    return ne, None


def _node_mlp_body(x_ref, a0_ref, a1_ref, w1a_ref, w1b_ref, b1_ref, w2_ref,
                   b2_ref, o_ref):
    h = jnp.maximum(
        _dot(x_ref[...], w1a_ref[...])
        + _dot(a0_ref[...] + a1_ref[...], w1b_ref[...])
        + b1_ref[...], 0.0)
    o_ref[...] = x_ref[...] + _dot(h, w2_ref[...]) + b2_ref[...]


def _node_mlp(x, agg0, agg1, w1a, w1b, b1, w2, b2, blk):
    n, d = x.shape
    dh = w1a.shape[1]
    do = w2.shape[1]
    return pl.pallas_call(
        _node_mlp_body,
        grid=(n // blk,),
        in_specs=[
            pl.BlockSpec((blk, d), lambda i: (i, 0)),
            pl.BlockSpec((blk, d), lambda i: (i, 0)),
            pl.BlockSpec((blk, d), lambda i: (i, 0)),
            pl.BlockSpec((d, dh), lambda i: (0, 0)),
            pl.BlockSpec((d, dh), lambda i: (0, 0)),
            pl.BlockSpec((1, dh), lambda i: (0, 0)),
            pl.BlockSpec((dh, do), lambda i: (0, 0)),
            pl.BlockSpec((1, do), lambda i: (0, 0)),
        ],
        out_specs=pl.BlockSpec((blk, do), lambda i: (i, 0)),
        out_shape=jax.ShapeDtypeStruct((n, do), F32),
    )(x, agg0, agg1, w1a, w1b, b1.reshape(1, dh), w2, b2.reshape(1, do))


def _combine_body(a_ref, ma_ref, b_ref, mb_ref, o_ref):
    o_ref[...] = a_ref[...] * ma_ref[...] + b_ref[...] * mb_ref[...]


def _combine(a, ma, b, mb, blk):
    n, d = a.shape
    return pl.pallas_call(
        _combine_body,
        grid=(n // blk,),
        in_specs=[
            pl.BlockSpec((blk, d), lambda i: (i, 0)),
            pl.BlockSpec((blk, 1), lambda i: (i, 0)),
            pl.BlockSpec((blk, d), lambda i: (i, 0)),
            pl.BlockSpec((blk, 1), lambda i: (i, 0)),
        ],
        out_specs=pl.BlockSpec((blk, d), lambda i: (i, 0)),
        out_shape=jax.ShapeDtypeStruct((n, d), F32),
    )(a, ma, b, mb)


def _final_body(ln_ref, pp_ref, wm_ref, pm_ref, p0_ref, w1_ref, b1_ref,
                w2_ref, b2_ref, o_ref):
    wm = wm_ref[...]
    pm = pm_ref[...]
    dec_in = ln_ref[...] * wm + pp_ref[...] * pm
    h = jnp.maximum(_dot(dec_in, w1_ref[...]) + b1_ref[...], 0.0)
    delta = _dot(h, w2_ref[...]) + b2_ref[...]
    o_ref[...] = (p0_ref[...] + delta) * (wm + pm)


def _final(ln, pp, wm, pm, p0, w1, b1, w2, b2, blk):
    n, d = ln.shape
    dh = w1.shape[1]
    do = w2.shape[1]
    return pl.pallas_call(
        _final_body,
        grid=(n // blk,),
        in_specs=[
            pl.BlockSpec((blk, d), lambda i: (i, 0)),
            pl.BlockSpec((blk, d), lambda i: (i, 0)),
            pl.BlockSpec((blk, 1), lambda i: (i, 0)),
            pl.BlockSpec((blk, 1), lambda i: (i, 0)),
            pl.BlockSpec((blk, 1), lambda i: (i, 0)),
            pl.BlockSpec((d, dh), lambda i: (0, 0)),
            pl.BlockSpec((1, dh), lambda i: (0, 0)),
            pl.BlockSpec((dh, do), lambda i: (0, 0)),
            pl.BlockSpec((1, do), lambda i: (0, 0)),
        ],
        out_specs=pl.BlockSpec((blk, do), lambda i: (i, 0)),
        out_shape=jax.ShapeDtypeStruct((n, do), F32),
    )(ln, pp, wm, pm, p0, w1, b1.reshape(1, dh), w2, b2.reshape(1, do))


def _gather_pair(table_s, table_r, s2d, r2d):
    nrows = s2d.shape[0]
    lat = table_s.shape[1]
    gch = 2
    inner = 2
    blkrows = gch * inner
    rows_pw = nrows // (_NC * _NS)
    nouter = rows_pw // blkrows
    cw = gch * _IB

    @functools.partial(
        pl.kernel,
        out_type=[jax.ShapeDtypeStruct((nrows * _IB, lat), F32),
                  jax.ShapeDtypeStruct((nrows * _IB, lat), F32)],
        mesh=plsc.VectorSubcoreMesh(core_axis_name="c", subcore_axis_name="s"),
        scratch_types=[pltpu.VMEM((blkrows, _IB), jnp.int32),
                       pltpu.VMEM((blkrows, _IB), jnp.int32),
                       pltpu.VMEM((cw, lat), F32),
                       pltpu.VMEM((cw, lat), F32),
                       pltpu.VMEM((cw, lat), F32),
                       pltpu.VMEM((cw, lat), F32),
                       pltpu.SemaphoreType.DMA,
                       pltpu.SemaphoreType.DMA],
        compiler_params=pltpu.CompilerParams(use_tc_tiling_on_sc=False),
    )
    def gk(ts_hbm, tr_hbm, s_hbm, r_hbm, outs_hbm, outr_hbm,
           sidx, ridx, bs0, br0, bs1, br1, sem0, sem1):
        wid = lax.axis_index("c") * _NS + lax.axis_index("s")
        row0 = wid * rows_pw
        bufs = [(bs0, br0, sem0), (bs1, br1, sem1)]

        def outer(o, carry):
            ro = row0 + o * blkrows
            pltpu.sync_copy(s_hbm.at[pl.ds(ro, blkrows)], sidx)
            pltpu.sync_copy(r_hbm.at[pl.ds(ro, blkrows)], ridx)

            def issue(c):
                bs, br, sem = bufs[c % 2]
                cps = []
                for k in range(gch):
                    cps.append(pltpu.async_copy(
                        ts_hbm.at[sidx.at[c * gch + k]],
                        bs.at[pl.ds(k * _IB, _IB)], sem))
                    cps.append(pltpu.async_copy(
                        tr_hbm.at[ridx.at[c * gch + k]],
                        br.at[pl.ds(k * _IB, _IB)], sem))
                return cps

            def drain(c, cps):
                bs, br, _ = bufs[c % 2]
                for cp in cps:
                    cp.wait()
                e0 = (ro + c * gch) * _IB
                pltpu.sync_copy(bs, outs_hbm.at[pl.ds(e0, cw)])
                pltpu.sync_copy(br, outr_hbm.at[pl.ds(e0, cw)])

            pend = [issue(0), issue(1)]
            for c in range(2, inner):
                drain(c - 2, pend[0])
                pend = [pend[1], issue(c)]
            drain(inner - 2, pend[0])
            drain(inner - 1, pend[1])
            return carry

        lax.fori_loop(0, nouter, outer, 0)

    return gk(table_s, table_r, s2d, r2d)


def _segment_sum_sc(ne, r2d, zeros, n_nodes, n_acc):
    nrows = r2d.shape[0]
    lat = ne.shape[1]
    half = lat // _NC
    sch = 4
    rows_ps = nrows // _NS
    nchunk = rows_ps // sch
    zrows = n_acc // _NS
    orows = n_nodes // _NS

    @functools.partial(
        pl.kernel,
        out_type=jax.ShapeDtypeStruct((n_nodes, lat), F32),
        mesh=plsc.VectorSubcoreMesh(core_axis_name="c", subcore_axis_name="s"),
        scratch_types=[pltpu.VMEM((sch, _IB), jnp.int32),
                       pltpu.VMEM((sch * _IB, half), F32),
                       pltpu.VMEM_SHARED((n_acc, half), F32)],
        compiler_params=pltpu.CompilerParams(use_tc_tiling_on_sc=False),
    )
    def sk(ne_hbm, r_hbm, z_hbm, out_hbm, ridx, vals, acc):
        c = lax.axis_index("c")
        s = lax.axis_index("s")
        col0 = c * half
        pltpu.sync_copy(z_hbm.at[pl.ds(s * zrows, zrows)],
                        acc.at[pl.ds(s * zrows, zrows)])
        plsc.subcore_barrier()

        def chunk(j, carry):
            r0 = s * rows_ps + j * sch
            e0 = r0 * _IB
            pltpu.sync_copy(r_hbm.at[pl.ds(r0, sch)], ridx)
            pltpu.sync_copy(ne_hbm.at[pl.ds(e0, sch * _IB), pl.ds(col0, half)],
                            vals)
            for k in range(sch):
                pltpu.sync_copy(vals.at[pl.ds(k * _IB, _IB)],
                                acc.at[ridx.at[k]], add=True)
            return carry

        lax.fori_loop(0, nchunk, chunk, 0)
        plsc.subcore_barrier()
        pltpu.sync_copy(acc.at[pl.ds(s * orows, orows)],
                        out_hbm.at[pl.ds(s * orows, orows),
                                   pl.ds(col0, half)])

    return sk(ne, r2d, zeros)


def _pad_idx(idx, pad_val, epad):
    pad = jnp.full((epad - idx.shape[0],), pad_val, jnp.int32)
    return jnp.concatenate([idx, pad]).reshape(epad // _IB, _IB)


def _pick_blk(n):
    for b in (2048, 2000, 1600, 1280, 1024, 800, 640, 512, 400, 320, 256,
              200, 160, 128):
        if n % b == 0:
            return b
    return n


def _gnn_steps(step_params, nodes, edge_halves, idx_halves, zeros, nblk,
               n_acc):
    n_nodes = nodes.shape[0]
    lat = nodes.shape[1]
    nsteps = len(step_params)
    for i, sp in enumerate(step_params):
        (w1e, b1e), (w2e, b2e) = sp['edge']
        (w1n, b1n), (w2n, b2n) = sp['node']
        gs, gr = _proj2(nodes, w1e[lat:2 * lat], w1e[2 * lat:], nblk, n_acc)
        gath = [_gather_pair(gs, gr, s2, r2) for (s2, r2) in idx_halves]
        want_res = i + 1 < nsteps
        aggs = []
        next_halves = []
        for h, e_h in enumerate(edge_halves):
            gsg, grg = gath[h]
            hpad = gsg.shape[0]
            ne, e_next = _edge_mlp(e_h, gsg, grg, w1e[:lat], b1e, w2e, b2e,
                                   _pick_blk(e_h.shape[0]),
                                   want_residual=want_res, epad=hpad)
            aggs.append(_segment_sum_sc(ne, idx_halves[h][1], zeros,
                                        n_nodes, n_acc))
            next_halves.append(e_next)
        nodes = _node_mlp(nodes, aggs[0], aggs[1], w1n[:lat], w1n[lat:],
                          b1n, w2n, b2n, nblk)
        edge_halves = next_halves
    return nodes


def kernel(nodes, edges, probe_nodes, probe_edges, wt_mask, probe_mask,
           params, senders, receivers, probe_senders, probe_receivers):
    n_nodes = nodes.shape[0]
    n_edges = edges.shape[0]
    lat = params['embed_node'][-1][0].shape[1]
    nblk = 2000 if n_nodes % 2000 == 0 else n_nodes
    align = 2 * _IB * _NC * _NS * 4
    epad = -(-n_edges // align) * align
    hpad = epad // 2
    n_acc = -(-(n_nodes + 1) // _NS) * _NS
    cut = min(hpad, n_edges)

    def halves_idx(s, r):
        s2 = _pad_idx(s, n_nodes, epad)
        r2 = _pad_idx(r, n_nodes, epad)
        hr = hpad // _IB
        return [(s2[:hr], r2[:hr]), (s2[hr:], r2[hr:])]

    idx_halves = halves_idx(senders, receivers)
    pidx_halves = halves_idx(probe_senders, probe_receivers)
    zeros = jnp.zeros((n_acc, lat // _NC), F32)

    (enw1, enb1), (enw2, enb2) = params['embed_node']
    (eew1, eeb1), (eew2, eeb2) = params['embed_edge']
    n_lat = _embed(nodes, enw1, enb1, enw2, enb2, nblk)

    def embed_edge_halves(e):
        return [_embed(e[:cut], eew1, eeb1, eew2, eeb2, _pick_blk(cut)),
                _embed(e[cut:], eew1, eeb1, eew2, eeb2,
                       _pick_blk(e.shape[0] - cut))]

    e_halves = embed_edge_halves(edges)
    pe_halves = embed_edge_halves(probe_edges)

    wt_out = _gnn_steps(params['wt_gnn'], n_lat, e_halves, idx_halves,
                        zeros, nblk, n_acc)
    latentspace = _combine(wt_out, wt_mask, n_lat, probe_mask, nblk)

    probe_out = _gnn_steps(params['probe_gnn'], latentspace, pe_halves,
                           pidx_halves, zeros, nblk, n_acc)

    (dw1, db1), (dw2, db2) = params['decoder']
    return _final(latentspace, probe_out, wt_mask, probe_mask,
                  probe_nodes[:, 0:1], dw1, db1, dw2, db2, nblk)
